# baseline (speedup 1.0000x reference)
import functools

import jax
import jax.numpy as jnp
from jax import lax
from jax.experimental import pallas as pl
from jax.experimental.pallas import tpu as pltpu
from jax.experimental.pallas import tpu_sc as plsc

N, E, D, DE, H = 10000, 320000, 128, 16, 128

_NT = 16
_B = 80
_EPT = E // _NT
_NBLK = _EPT // _B
_ACC_ROWS = 10240
_TRASH = N

_NB_N = 10
_BN = N // _NB_N
_NB_E = 40
_BE = E // _NB_E


def _proj_nodes_body(x_ref, w_ref, o_ref):
    o_ref[0] = jnp.dot(x_ref[...], w_ref[0], preferred_element_type=jnp.float32)


def _proj_nodes(x, wx):
    return pl.pallas_call(
        _proj_nodes_body,
        grid=(2, _NB_N),
        in_specs=[
            pl.BlockSpec((_BN, D), lambda c, n: (n, 0)),
            pl.BlockSpec((1, D, H), lambda c, n: (c, 0, 0)),
        ],
        out_specs=pl.BlockSpec((1, _BN, H), lambda c, n: (c, n, 0)),
        out_shape=jax.ShapeDtypeStruct((2, N, H), jnp.float32),
    )(x, wx)


def _proj_edges_body(ea_ref, w_ref, b_ref, o_ref):
    o_ref[0] = (
        jnp.dot(ea_ref[...], w_ref[0], preferred_element_type=jnp.float32)
        + b_ref[0]
    ).astype(jnp.bfloat16)


def _proj_edges(ea, we, bcat):
    return pl.pallas_call(
        _proj_edges_body,
        grid=(2, _NB_E),
        in_specs=[
            pl.BlockSpec((_BE, DE), lambda c, e: (e, 0)),
            pl.BlockSpec((1, DE, H), lambda c, e: (c, 0, 0)),
            pl.BlockSpec((1, 1, H), lambda c, e: (c, 0, 0)),
        ],
        out_specs=pl.BlockSpec((1, _BE, H), lambda c, e: (c, e, 0)),
        out_shape=jax.ShapeDtypeStruct((2, E, H), jnp.bfloat16),
    )(ea, we, bcat)


def _sc_flow_body(row_hbm, col_hbm, p_hbm, q_hbm, out_hbm,
                  row_v0, col_v0, idx_v0, dst_v0, row_v1, col_v1,
                  idx_v1, dst_v1, sdst_v, pbuf0, qbuf0, pbuf1, qbuf1,
                  hbuf, zbuf, acc,
                  rcsem0, rcsem1, gqsem0, gqsem1, ssem):
    c = lax.axis_index("c")
    t = lax.axis_index("s")
    row_v = (row_v0, row_v1)
    col_v = (col_v0, col_v1)
    idx_v = (idx_v0, idx_v1)
    dst_v = (dst_v0, dst_v1)
    pbuf = (pbuf0, pbuf1)
    qbuf = (qbuf0, qbuf1)
    rcsem = (rcsem0, rcsem1)
    gqsem = (gqsem0, gqsem1)

    zero = jnp.zeros((16,), jnp.float32)
    for i in range(8):
        for j in range(H // 16):
            zbuf[i, pl.ds(j * 16, 16)] = zero

    def _zero_acc(i, carry):
        pltpu.sync_copy(zbuf, acc.at[pl.ds(t * (_ACC_ROWS // _NT) + i * 8, 8)])
        return carry

    lax.fori_loop(0, _ACC_ROWS // _NT // 8, _zero_acc, 0)
    plsc.subcore_barrier()

    sign = 1 - 2 * c
    base0 = t * _EPT
    qoff = c * E
    trash = _TRASH + t

    def rc_issue(b, s):
        pltpu.async_copy(row_hbm.at[pl.ds(base0 + b * _B, _B)], row_v[s], rcsem[s])
        pltpu.async_copy(col_hbm.at[pl.ds(base0 + b * _B, _B)], col_v[s], rcsem[s])

    def rc_wait(s):
        pltpu.make_async_copy(row_hbm.at[pl.ds(0, _B)], row_v[s], rcsem[s]).wait()
        pltpu.make_async_copy(col_hbm.at[pl.ds(0, _B)], col_v[s], rcsem[s]).wait()

    def idx_compute(s):
        for i in range(_B // 16):
            sl = pl.ds(i * 16, 16)
            r = row_v[s][sl]
            cl = col_v[s][sl]
            keep = ((cl - r) * sign) > 0
            dst_v[s][sl] = jnp.where(keep, r, trash)
            idx_v[s][sl] = cl + c * N

    def gq_issue(b, s):
        pltpu.async_copy(p_hbm.at[idx_v[s]], pbuf[s], gqsem[s])
        pltpu.async_copy(
            q_hbm.at[pl.ds((qoff + base0 + b * _B) * (H // 2), _B * (H // 2))],
            qbuf[s], gqsem[s])

    def gq_wait(s):
        pltpu.make_async_copy(p_hbm.at[idx_v[s]], pbuf[s], gqsem[s]).wait()
        pltpu.make_async_copy(q_hbm.at[pl.ds(0, _B * (H // 2))], qbuf[s],
                              gqsem[s]).wait()

    zero_f = jnp.zeros((16,), jnp.float32)
    himask = jnp.full((16,), -65536, jnp.int32)

    def relu(s):
        def _relu_row(i, carry2):
            for j in range(H // 32):
                qi = qbuf[s][pl.ds(i * (H // 2) + j * 16, 16)]
                qa = lax.bitcast_convert_type(qi << 16, jnp.float32)
                qb = lax.bitcast_convert_type(qi & himask, jnp.float32)
                va = pbuf[s][i, pl.ds(j * 32, 16)] + qa
                vb = pbuf[s][i, pl.ds(j * 32 + 16, 16)] + qb
                hbuf[i, pl.ds(j * 32, 16)] = jnp.maximum(va, zero_f)
                hbuf[i, pl.ds(j * 32 + 16, 16)] = jnp.maximum(vb, zero_f)
            return carry2

        lax.fori_loop(0, _B, _relu_row, 0)

    def scat_issue(s):
        for i in range(_B // 16):
            sl = pl.ds(i * 16, 16)
            sdst_v[sl] = dst_v[s][sl]
        pltpu.async_copy(hbuf, acc.at[sdst_v], ssem, add=True)

    def scat_wait():
        pltpu.make_async_copy(hbuf, acc.at[sdst_v], ssem).wait()

    rc_issue(0, 0)
    rc_wait(0)
    idx_compute(0)
    gq_issue(0, 0)
    rc_issue(1, 1)

    def _iter(g, carry):
        b = 2 * g
        rc_wait(1)
        idx_compute(1)
        gq_issue(b + 1, 1)

        @pl.when(g < _NBLK // 2 - 1)
        def _():
            rc_issue(b + 2, 0)

        gq_wait(0)

        @pl.when(g >= 1)
        def _():
            scat_wait()

        relu(0)
        scat_issue(0)

        @pl.when(g < _NBLK // 2 - 1)
        def _():
            rc_wait(0)
            idx_compute(0)
            gq_issue(b + 2, 0)
            rc_issue(b + 3, 1)

        gq_wait(1)
        scat_wait()
        relu(1)
        scat_issue(1)
        return carry

    lax.fori_loop(0, _NBLK // 2, _iter, 0)
    scat_wait()
    plsc.subcore_barrier()

    @pl.when(t < _NT - 1)
    def _():
        pltpu.sync_copy(acc.at[pl.ds(t * 640, 640)],
                        out_hbm.at[c, pl.ds(t * 640, 640)])

    @pl.when(t == _NT - 1)
    def _():
        pltpu.sync_copy(acc.at[pl.ds(9600, 400)],
                        out_hbm.at[c, pl.ds(9600, 400)])


def _sc_flow(row, col, p2, q2):
    mesh = plsc.VectorSubcoreMesh(core_axis_name="c", subcore_axis_name="s")
    f = functools.partial(
        pl.kernel,
        mesh=mesh,
        out_type=jax.ShapeDtypeStruct((2, N, H), jnp.float32),
        scratch_types=(
            [pltpu.VMEM((_B,), jnp.int32)] * 9
            + [pltpu.VMEM((_B, H), jnp.float32),
               pltpu.VMEM((_B * H // 2,), jnp.int32)] * 2
            + [pltpu.VMEM((_B, H), jnp.float32),
               pltpu.VMEM((8, H), jnp.float32),
               pltpu.VMEM_SHARED((_ACC_ROWS, H), jnp.float32)]
            + [pltpu.SemaphoreType.DMA] * 5
        ),
    )(_sc_flow_body)
    return f(row, col, p2, q2)


def _node_mlp_body(fi_ref, fo_ref, wi_ref, wo_ref, b_ref, o_ref):
    acc = jnp.dot(fi_ref[...], wi_ref[...], preferred_element_type=jnp.float32)
    acc += jnp.dot(fo_ref[...], wo_ref[...], preferred_element_type=jnp.float32)
    o_ref[...] = jnp.maximum(acc + b_ref[...], 0.0)


def _node_mlp(fi, fo, wi, wo, bn):
    return pl.pallas_call(
        _node_mlp_body,
        grid=(_NB_N,),
        in_specs=[
            pl.BlockSpec((_BN, H), lambda n: (n, 0)),
            pl.BlockSpec((_BN, H), lambda n: (n, 0)),
            pl.BlockSpec((H, H), lambda n: (0, 0)),
            pl.BlockSpec((H, H), lambda n: (0, 0)),
            pl.BlockSpec((1, H), lambda n: (0, 0)),
        ],
        out_specs=pl.BlockSpec((_BN, H), lambda n: (n, 0)),
        out_shape=jax.ShapeDtypeStruct((N, H), jnp.float32),
    )(fi, fo, wi, wo, bn)


def kernel(x, edge_attr, W_in, b_in, W_out, b_out, W_node, b_node, edge_index):
    row = edge_index[0]
    col = edge_index[1]
    perm = jnp.asarray(
        [32 * g + 2 * k + h for g in range(H // 32)
         for h in range(2) for k in range(16)], dtype=jnp.int32)
    wx = jnp.stack([W_out[:D], W_in[:D]])[:, :, perm]
    we = jnp.stack([W_out[D:], W_in[D:]])
    bcat = jnp.stack([b_out, b_in])[:, None, :]

    p = _proj_nodes(x, wx)
    q = _proj_edges(edge_attr, we, bcat)
    q2 = jax.lax.bitcast_convert_type(
        q.reshape(2 * E, H // 2, 2), jnp.int32).reshape(2 * E * (H // 2))
    flow = _sc_flow(row, col, p.reshape(2 * N, H), q2)
    f_o, f_i = flow[0], flow[1]
    wn_i = jnp.take(W_node[:H], perm, axis=0)
    wn_o = jnp.take(W_node[H:], perm, axis=0)
    return _node_mlp(f_i, f_o, wn_i, wn_o, b_node[None, :])


# speedup vs baseline: 4.2453x; 4.2453x over previous
import functools

import jax
import jax.numpy as jnp
from jax import lax
from jax.experimental import pallas as pl
from jax.experimental.pallas import tpu as pltpu
from jax.experimental.pallas import tpu_sc as plsc

N, E, D, DE, H = 10000, 320000, 128, 16, 128

_NT = 16
_B = 80
_EPT = E // _NT
_NBLK = _EPT // _B
_ACC_ROWS = 10240
_TRASH = N

_NB_N = 10
_BN = N // _NB_N
_NB_E = 40
_BE = E // _NB_E


def _proj_nodes_body(x_ref, w_ref, o_ref):
    o_ref[0] = jnp.dot(x_ref[...], w_ref[0], preferred_element_type=jnp.float32)


def _proj_nodes(x, wx):
    return pl.pallas_call(
        _proj_nodes_body,
        grid=(2, _NB_N),
        in_specs=[
            pl.BlockSpec((_BN, D), lambda c, n: (n, 0)),
            pl.BlockSpec((1, D, H), lambda c, n: (c, 0, 0)),
        ],
        out_specs=pl.BlockSpec((1, _BN, H), lambda c, n: (c, n, 0)),
        out_shape=jax.ShapeDtypeStruct((2, N, H), jnp.float32),
    )(x, wx)


def _proj_edges_body(ea_ref, w_ref, b_ref, o_ref):
    y = (jnp.dot(ea_ref[...], w_ref[0], preferred_element_type=jnp.float32)
         + b_ref[0])
    yi = lax.bitcast_convert_type(y, jnp.int32)
    rb = (yi + 0x7FFF + ((yi >> 16) & 1)) >> 16
    o_ref[0] = (rb[: _BE // 2] & 0xFFFF) | (rb[_BE // 2:] << 16)


def _proj_edges(ea, we, bcat):
    return pl.pallas_call(
        _proj_edges_body,
        grid=(2, _NB_E),
        in_specs=[
            pl.BlockSpec((_BE, DE), lambda c, e: (e, 0)),
            pl.BlockSpec((1, DE, H), lambda c, e: (c, 0, 0)),
            pl.BlockSpec((1, 1, H), lambda c, e: (c, 0, 0)),
        ],
        out_specs=pl.BlockSpec((1, _BE // 2, H), lambda c, e: (c, e, 0)),
        out_shape=jax.ShapeDtypeStruct((2, E // 2, H), jnp.int32),
    )(ea, we, bcat)


def _sc_flow_body(row_hbm, col_hbm, p_hbm, q_hbm, out_hbm,
                  row_v0, col_v0, idx_v0, dst_v0, row_v1, col_v1,
                  idx_v1, dst_v1, sdst_v, pbuf0, qbuf0, pbuf1, qbuf1,
                  zbuf, acc,
                  rcsem0, rcsem1, gqsem0, gqsem1, ssem):
    c = lax.axis_index("c")
    t = lax.axis_index("s")
    row_v = (row_v0, row_v1)
    col_v = (col_v0, col_v1)
    idx_v = (idx_v0, idx_v1)
    dst_v = (dst_v0, dst_v1)
    pbuf = (pbuf0, pbuf1)
    qbuf = (qbuf0, qbuf1)
    rcsem = (rcsem0, rcsem1)
    gqsem = (gqsem0, gqsem1)

    zero = jnp.zeros((16,), jnp.float32)
    for i in range(8):
        for j in range(H // 16):
            zbuf[i, pl.ds(j * 16, 16)] = zero

    def _zero_acc(i, carry):
        pltpu.sync_copy(zbuf, acc.at[pl.ds(t * (_ACC_ROWS // _NT) + i * 8, 8)])
        return carry

    lax.fori_loop(0, _ACC_ROWS // _NT // 8, _zero_acc, 0)
    plsc.subcore_barrier()

    sign = 1 - 2 * c
    base0 = t * _EPT
    qoff = c * E
    trash = _TRASH + t

    def rc_issue(b, s):
        pltpu.async_copy(row_hbm.at[pl.ds(base0 + b * _B, _B)], row_v[s], rcsem[s])
        pltpu.async_copy(col_hbm.at[pl.ds(base0 + b * _B, _B)], col_v[s], rcsem[s])

    def rc_wait(s):
        pltpu.make_async_copy(row_hbm.at[pl.ds(0, _B)], row_v[s], rcsem[s]).wait()
        pltpu.make_async_copy(col_hbm.at[pl.ds(0, _B)], col_v[s], rcsem[s]).wait()

    def idx_compute(s):
        for i in range(_B // 16):
            sl = pl.ds(i * 16, 16)
            r = row_v[s][sl]
            cl = col_v[s][sl]
            keep = ((cl - r) * sign) > 0
            dst_v[s][sl] = jnp.where(keep, r, trash)
            idx_v[s][sl] = cl + c * N

    def _qrow(b):
        e = base0 + b * _B
        blk = e // _BE
        r = e - blk * _BE
        half = r // (_BE // 2)
        pr = r - half * (_BE // 2)
        prow = pl.multiple_of(qoff // 2 + blk * (_BE // 2) + pr, _B)
        return prow, half

    def gq_issue(b, s):
        pltpu.async_copy(p_hbm.at[idx_v[s]], pbuf[s], gqsem[s])
        prow, _ = _qrow(b)
        pltpu.async_copy(q_hbm.at[pl.ds(prow, _B)], qbuf[s], gqsem[s])

    def gq_wait(s):
        pltpu.make_async_copy(p_hbm.at[idx_v[s]], pbuf[s], gqsem[s]).wait()
        pltpu.make_async_copy(q_hbm.at[pl.ds(0, _B)], qbuf[s],
                              gqsem[s]).wait()

    zero_f = jnp.zeros((16,), jnp.float32)
    himask = jnp.full((16,), -65536, jnp.int32)

    def relu(s, b):
        _, half = _qrow(b)
        sh = 16 * half

        def _relu_row(i, carry2):
            for j in range(H // 16):
                sj = pl.ds(j * 16, 16)
                qi = lax.shift_right_logical(qbuf[s][i, sj], sh)
                qv = lax.bitcast_convert_type(qi << 16, jnp.float32)
                pbuf[s][i, sj] = jnp.maximum(pbuf[s][i, sj] + qv, zero_f)
            return carry2

        lax.fori_loop(0, _B, _relu_row, 0)

    def scat_issue(s):
        for i in range(_B // 16):
            sl = pl.ds(i * 16, 16)
            sdst_v[sl] = dst_v[s][sl]
        pltpu.async_copy(pbuf[s], acc.at[sdst_v], ssem, add=True)

    def scat_wait():
        pltpu.make_async_copy(pbuf[0], acc.at[sdst_v], ssem).wait()

    rc_issue(0, 0)
    rc_wait(0)
    idx_compute(0)
    gq_issue(0, 0)
    rc_issue(1, 1)

    def _iter(g, carry):
        b = 2 * g
        rc_wait(1)

        @pl.when(g >= 1)
        def _():
            scat_wait()

        idx_compute(1)
        gq_issue(b + 1, 1)

        @pl.when(g < _NBLK // 2 - 1)
        def _():
            rc_issue(b + 2, 0)

        gq_wait(0)
        relu(0, b)
        scat_issue(0)

        scat_wait()

        @pl.when(g < _NBLK // 2 - 1)
        def _():
            rc_wait(0)
            idx_compute(0)
            gq_issue(b + 2, 0)
            rc_issue(b + 3, 1)

        gq_wait(1)
        relu(1, b + 1)
        scat_issue(1)
        return carry

    lax.fori_loop(0, _NBLK // 2, _iter, 0)
    scat_wait()
    plsc.subcore_barrier()

    @pl.when(t < _NT - 1)
    def _():
        pltpu.sync_copy(acc.at[pl.ds(t * 640, 640)],
                        out_hbm.at[c, pl.ds(t * 640, 640)])

    @pl.when(t == _NT - 1)
    def _():
        pltpu.sync_copy(acc.at[pl.ds(9600, 400)],
                        out_hbm.at[c, pl.ds(9600, 400)])


def _sc_flow(row, col, p2, q2):
    mesh = plsc.VectorSubcoreMesh(core_axis_name="c", subcore_axis_name="s")
    f = functools.partial(
        pl.kernel,
        mesh=mesh,
        out_type=jax.ShapeDtypeStruct((2, N, H), jnp.float32),
        scratch_types=(
            [pltpu.VMEM((_B,), jnp.int32)] * 9
            + [pltpu.VMEM((_B, H), jnp.float32),
               pltpu.VMEM((_B, H), jnp.int32)] * 2
            + [pltpu.VMEM((8, H), jnp.float32),
               pltpu.VMEM_SHARED((_ACC_ROWS, H), jnp.float32)]
            + [pltpu.SemaphoreType.DMA] * 5
        ),
    )(_sc_flow_body)
    return f(row, col, p2, q2)


def _node_mlp_body(fi_ref, fo_ref, wi_ref, wo_ref, b_ref, o_ref):
    acc = jnp.dot(fi_ref[...], wi_ref[...], preferred_element_type=jnp.float32)
    acc += jnp.dot(fo_ref[...], wo_ref[...], preferred_element_type=jnp.float32)
    o_ref[...] = jnp.maximum(acc + b_ref[...], 0.0)


def _node_mlp(fi, fo, wi, wo, bn):
    return pl.pallas_call(
        _node_mlp_body,
        grid=(_NB_N,),
        in_specs=[
            pl.BlockSpec((_BN, H), lambda n: (n, 0)),
            pl.BlockSpec((_BN, H), lambda n: (n, 0)),
            pl.BlockSpec((H, H), lambda n: (0, 0)),
            pl.BlockSpec((H, H), lambda n: (0, 0)),
            pl.BlockSpec((1, H), lambda n: (0, 0)),
        ],
        out_specs=pl.BlockSpec((_BN, H), lambda n: (n, 0)),
        out_shape=jax.ShapeDtypeStruct((N, H), jnp.float32),
    )(fi, fo, wi, wo, bn)


def kernel(x, edge_attr, W_in, b_in, W_out, b_out, W_node, b_node, edge_index):
    row = edge_index[0]
    col = edge_index[1]
    wx = jnp.stack([W_out[:D], W_in[:D]])
    we = jnp.stack([W_out[D:], W_in[D:]])
    bcat = jnp.stack([b_out, b_in])[:, None, :]

    p = _proj_nodes(x, wx)
    q = _proj_edges(edge_attr, we, bcat)
    flow = _sc_flow(row, col, p.reshape(2 * N, H), q.reshape(E, H))
    f_o, f_i = flow[0], flow[1]
    return _node_mlp(f_i, f_o, W_node[:H], W_node[H:], b_node[None, :])


# speedup vs baseline: 4.2945x; 1.0116x over previous
import functools

import jax
import jax.numpy as jnp
from jax import lax
from jax.experimental import pallas as pl
from jax.experimental.pallas import tpu as pltpu
from jax.experimental.pallas import tpu_sc as plsc

N, E, D, DE, H = 10000, 320000, 128, 16, 128

_NT = 16
_B = 80
_EPT = E // _NT
_NBLK = _EPT // _B
_ACC_ROWS = 10240
_TRASH = N

_NB_N = 10
_BN = N // _NB_N
_NB_E = 40
_BE = E // _NB_E


def _proj_nodes_body(x_ref, w_ref, o_ref):
    o_ref[0] = jnp.dot(x_ref[...], w_ref[0], preferred_element_type=jnp.float32)


def _proj_nodes(x, wx):
    return pl.pallas_call(
        _proj_nodes_body,
        grid=(2, _NB_N),
        in_specs=[
            pl.BlockSpec((_BN, D), lambda c, n: (n, 0)),
            pl.BlockSpec((1, D, H), lambda c, n: (c, 0, 0)),
        ],
        out_specs=pl.BlockSpec((1, _BN, H), lambda c, n: (c, n, 0)),
        out_shape=jax.ShapeDtypeStruct((2, N, H), jnp.float32),
    )(x, wx)


def _proj_edges_body(ea_ref, w_ref, b_ref, o_ref):
    y = (jnp.dot(ea_ref[...], w_ref[0], preferred_element_type=jnp.float32)
         + b_ref[0])
    yi = lax.bitcast_convert_type(y, jnp.int32)
    rb = (yi + 0x7FFF + ((yi >> 16) & 1)) >> 16
    o_ref[0] = (rb[: _BE // 2] & 0xFFFF) | (rb[_BE // 2:] << 16)


def _proj_edges(ea, we, bcat):
    return pl.pallas_call(
        _proj_edges_body,
        grid=(2, _NB_E),
        in_specs=[
            pl.BlockSpec((_BE, DE), lambda c, e: (e, 0)),
            pl.BlockSpec((1, DE, H), lambda c, e: (c, 0, 0)),
            pl.BlockSpec((1, 1, H), lambda c, e: (c, 0, 0)),
        ],
        out_specs=pl.BlockSpec((1, _BE // 2, H), lambda c, e: (c, e, 0)),
        out_shape=jax.ShapeDtypeStruct((2, E // 2, H), jnp.int32),
    )(ea, we, bcat)


def _sc_flow_body(row_hbm, col_hbm, p_hbm, q_hbm, out_hbm,
                  row_v0, col_v0, idx_v0, dst_v0, row_v1, col_v1,
                  idx_v1, dst_v1, sdst_v, pbuf0, qbuf0, pbuf1, qbuf1,
                  zbuf, acc,
                  rcsem0, rcsem1, gqsem0, gqsem1, ssem):
    c = lax.axis_index("c")
    t = lax.axis_index("s")
    row_v = (row_v0, row_v1)
    col_v = (col_v0, col_v1)
    idx_v = (idx_v0, idx_v1)
    dst_v = (dst_v0, dst_v1)
    pbuf = (pbuf0, pbuf1)
    qbuf = (qbuf0, qbuf1)
    rcsem = (rcsem0, rcsem1)
    gqsem = (gqsem0, gqsem1)

    zero = jnp.zeros((16,), jnp.float32)
    for i in range(8):
        for j in range(H // 16):
            zbuf[i, pl.ds(j * 16, 16)] = zero

    def _zero_acc(i, carry):
        pltpu.sync_copy(zbuf, acc.at[pl.ds(t * (_ACC_ROWS // _NT) + i * 8, 8)])
        return carry

    lax.fori_loop(0, _ACC_ROWS // _NT // 8, _zero_acc, 0)
    plsc.subcore_barrier()

    sign = 1 - 2 * c
    base0 = t * _EPT
    qoff = c * E
    trash = _TRASH + t

    def rc_issue(b, s):
        pltpu.async_copy(row_hbm.at[pl.ds(base0 + b * _B, _B)], row_v[s], rcsem[s])
        pltpu.async_copy(col_hbm.at[pl.ds(base0 + b * _B, _B)], col_v[s], rcsem[s])

    def rc_wait(s):
        pltpu.make_async_copy(row_hbm.at[pl.ds(0, _B)], row_v[s], rcsem[s]).wait()
        pltpu.make_async_copy(col_hbm.at[pl.ds(0, _B)], col_v[s], rcsem[s]).wait()

    def idx_compute(s):
        for i in range(_B // 16):
            sl = pl.ds(i * 16, 16)
            r = row_v[s][sl]
            cl = col_v[s][sl]
            keep = ((cl - r) * sign) > 0
            dst_v[s][sl] = jnp.where(keep, r, trash)
            idx_v[s][sl] = cl + c * N

    def _qrow(b):
        e = base0 + b * _B
        blk = e // _BE
        r = e - blk * _BE
        half = r // (_BE // 2)
        pr = r - half * (_BE // 2)
        prow = pl.multiple_of(qoff // 2 + blk * (_BE // 2) + pr, _B)
        return prow, half

    def gq_issue(b, s):
        pltpu.async_copy(p_hbm.at[idx_v[s]], pbuf[s], gqsem[s])
        prow, _ = _qrow(b)
        pltpu.async_copy(q_hbm.at[pl.ds(prow, _B)], qbuf[s], gqsem[s])

    def gq_wait(s):
        pltpu.make_async_copy(p_hbm.at[idx_v[s]], pbuf[s], gqsem[s]).wait()
        pltpu.make_async_copy(q_hbm.at[pl.ds(0, _B)], qbuf[s],
                              gqsem[s]).wait()

    zero_f = jnp.zeros((16,), jnp.float32)
    himask = jnp.full((16,), -65536, jnp.int32)

    def relu(s, b):
        _, half = _qrow(b)
        sh = 16 * half

        def _relu_row(i4, carry2):
            for u in range(4):
                i = i4 * 4 + u
                for j in range(H // 16):
                    sj = pl.ds(j * 16, 16)
                    qi = lax.shift_right_logical(qbuf[s][i, sj], sh)
                    qv = lax.bitcast_convert_type(qi << 16, jnp.float32)
                    pbuf[s][i, sj] = jnp.maximum(pbuf[s][i, sj] + qv, zero_f)
            return carry2

        lax.fori_loop(0, _B // 4, _relu_row, 0)

    def scat_issue(s):
        for i in range(_B // 16):
            sl = pl.ds(i * 16, 16)
            sdst_v[sl] = dst_v[s][sl]
        pltpu.async_copy(pbuf[s], acc.at[sdst_v], ssem, add=True)

    def scat_wait():
        pltpu.make_async_copy(pbuf[0], acc.at[sdst_v], ssem).wait()

    rc_issue(0, 0)
    rc_wait(0)
    idx_compute(0)
    gq_issue(0, 0)
    rc_issue(1, 1)

    def _iter(g, carry):
        b = 2 * g
        rc_wait(1)

        @pl.when(g >= 1)
        def _():
            scat_wait()

        idx_compute(1)
        gq_issue(b + 1, 1)

        @pl.when(g < _NBLK // 2 - 1)
        def _():
            rc_issue(b + 2, 0)

        gq_wait(0)
        relu(0, b)
        scat_issue(0)

        scat_wait()

        @pl.when(g < _NBLK // 2 - 1)
        def _():
            rc_wait(0)
            idx_compute(0)
            gq_issue(b + 2, 0)
            rc_issue(b + 3, 1)

        gq_wait(1)
        relu(1, b + 1)
        scat_issue(1)
        return carry

    lax.fori_loop(0, _NBLK // 2, _iter, 0)
    scat_wait()
    plsc.subcore_barrier()

    @pl.when(t < _NT - 1)
    def _():
        pltpu.sync_copy(acc.at[pl.ds(t * 640, 640)],
                        out_hbm.at[c, pl.ds(t * 640, 640)])

    @pl.when(t == _NT - 1)
    def _():
        pltpu.sync_copy(acc.at[pl.ds(9600, 400)],
                        out_hbm.at[c, pl.ds(9600, 400)])


def _sc_flow(row, col, p2, q2):
    mesh = plsc.VectorSubcoreMesh(core_axis_name="c", subcore_axis_name="s")
    f = functools.partial(
        pl.kernel,
        mesh=mesh,
        out_type=jax.ShapeDtypeStruct((2, N, H), jnp.float32),
        scratch_types=(
            [pltpu.VMEM((_B,), jnp.int32)] * 9
            + [pltpu.VMEM((_B, H), jnp.float32),
               pltpu.VMEM((_B, H), jnp.int32)] * 2
            + [pltpu.VMEM((8, H), jnp.float32),
               pltpu.VMEM_SHARED((_ACC_ROWS, H), jnp.float32)]
            + [pltpu.SemaphoreType.DMA] * 5
        ),
    )(_sc_flow_body)
    return f(row, col, p2, q2)


def _node_mlp_body(fi_ref, fo_ref, wi_ref, wo_ref, b_ref, o_ref):
    acc = jnp.dot(fi_ref[...], wi_ref[...], preferred_element_type=jnp.float32)
    acc += jnp.dot(fo_ref[...], wo_ref[...], preferred_element_type=jnp.float32)
    o_ref[...] = jnp.maximum(acc + b_ref[...], 0.0)


def _node_mlp(fi, fo, wi, wo, bn):
    return pl.pallas_call(
        _node_mlp_body,
        grid=(_NB_N,),
        in_specs=[
            pl.BlockSpec((_BN, H), lambda n: (n, 0)),
            pl.BlockSpec((_BN, H), lambda n: (n, 0)),
            pl.BlockSpec((H, H), lambda n: (0, 0)),
            pl.BlockSpec((H, H), lambda n: (0, 0)),
            pl.BlockSpec((1, H), lambda n: (0, 0)),
        ],
        out_specs=pl.BlockSpec((_BN, H), lambda n: (n, 0)),
        out_shape=jax.ShapeDtypeStruct((N, H), jnp.float32),
    )(fi, fo, wi, wo, bn)


def kernel(x, edge_attr, W_in, b_in, W_out, b_out, W_node, b_node, edge_index):
    row = edge_index[0]
    col = edge_index[1]
    wx = jnp.stack([W_out[:D], W_in[:D]])
    we = jnp.stack([W_out[D:], W_in[D:]])
    bcat = jnp.stack([b_out, b_in])[:, None, :]

    p = _proj_nodes(x, wx)
    q = _proj_edges(edge_attr, we, bcat)
    flow = _sc_flow(row, col, p.reshape(2 * N, H), q.reshape(E, H))
    f_o, f_i = flow[0], flow[1]
    return _node_mlp(f_i, f_o, W_node[:H], W_node[H:], b_node[None, :])
